# baseline (speedup 1.0000x reference)
import jax
import jax.numpy as jnp
from jax.experimental import pallas as pl
from jax.experimental.pallas import tpu as pltpu

_SEQ_LENS = (2048, 1792, 1536, 1280, 1024, 896, 768, 512)
_D = 1024
_MAX_L = 2048
_MAX_PAD = _MAX_L - min(_SEQ_LENS)


def _body(*refs):
    in_refs = refs[:8]
    out_ref = refs[8]
    zeros_ref = refs[9]
    sem = refs[10]

    zeros_ref[...] = jnp.zeros(zeros_ref.shape, zeros_ref.dtype)

    copies = []
    for i, L in enumerate(_SEQ_LENS):
        copies.append(pltpu.make_async_copy(
            in_refs[i],
            out_ref.at[pl.ds(i, 1), pl.ds(0, L), :],
            sem,
        ))
        pad = _MAX_L - L
        if pad:
            copies.append(pltpu.make_async_copy(
                zeros_ref.at[:, pl.ds(0, pad), :],
                out_ref.at[pl.ds(i, 1), pl.ds(L, pad), :],
                sem,
            ))
    for c in copies:
        c.start()
    for c in copies:
        c.wait()


def kernel(seq0, seq1, seq2, seq3, seq4, seq5, seq6, seq7):
    seqs = (seq0, seq1, seq2, seq3, seq4, seq5, seq6, seq7)
    out_shape = jax.ShapeDtypeStruct((8, _MAX_L, _D), seq0.dtype)
    return pl.pallas_call(
        _body,
        in_specs=[pl.BlockSpec(memory_space=pl.ANY)] * 8,
        out_specs=pl.BlockSpec(memory_space=pl.ANY),
        out_shape=out_shape,
        scratch_shapes=[
            pltpu.VMEM((1, _MAX_PAD, _D), jnp.float32),
            pltpu.SemaphoreType.DMA,
        ],
    )(*seqs)


# speedup vs baseline: 25.3512x; 25.3512x over previous
import jax
import jax.numpy as jnp
from jax import lax
from jax.experimental import pallas as pl

_SEQ_LENS = (2048, 1792, 1536, 1280, 1024, 896, 768, 512)
_D = 1024
_TILE = 512
_MAX_L = 2048
_GRID_T = _MAX_L // _TILE
_N_TILES = tuple(-(-L // _TILE) for L in _SEQ_LENS)


def _body(*refs):
    in_refs = refs[:8]
    out_ref = refs[8]
    i = pl.program_id(0)
    t = pl.program_id(1)
    row = lax.broadcasted_iota(jnp.int32, (1, _TILE, _D), 1)
    for k, L in enumerate(_SEQ_LENS):
        @pl.when(i == k)
        def _(k=k, L=L):
            valid = L - t * _TILE
            out_ref[...] = jnp.where(row < valid, in_refs[k][...], 0.0)


def _in_spec(k):
    nk = _N_TILES[k]

    def index_map(s, t):
        tt = jnp.where(s == k, jnp.minimum(t, nk - 1), 0)
        return (0, tt, 0)

    return pl.BlockSpec((1, _TILE, _D), index_map)


def kernel(seq0, seq1, seq2, seq3, seq4, seq5, seq6, seq7):
    seqs = (seq0, seq1, seq2, seq3, seq4, seq5, seq6, seq7)
    out_shape = jax.ShapeDtypeStruct((8, _MAX_L, _D), seq0.dtype)
    return pl.pallas_call(
        _body,
        grid=(8, _GRID_T),
        in_specs=[_in_spec(k) for k in range(8)],
        out_specs=pl.BlockSpec((1, _TILE, _D), lambda s, t: (s, t, 0)),
        out_shape=out_shape,
    )(*seqs)


# speedup vs baseline: 37.1912x; 1.4670x over previous
import jax
import jax.numpy as jnp
from jax.experimental import pallas as pl
from jax.experimental.pallas import tpu as pltpu

_SEQ_LENS = (2048, 1792, 1536, 1280, 1024, 896, 768, 512)
_D = 1024
_MAX_L = 2048
_MAX_PAD = _MAX_L - min(_SEQ_LENS)


def _body(*refs):
    in_refs = refs[:8]
    out_ref = refs[8]
    bufs = refs[9:17]
    zero_ref = refs[17]
    in_sems = refs[18]
    out_sems = refs[19]
    pad_sems = refs[20]

    zero_ref[...] = jnp.zeros(zero_ref.shape, zero_ref.dtype)

    in_copies = [
        pltpu.make_async_copy(in_refs[i], bufs[i], in_sems.at[i])
        for i in range(8)
    ]
    for c in in_copies:
        c.start()

    pad_copies = []
    for i, L in enumerate(_SEQ_LENS):
        pad = _MAX_L - L
        if pad:
            c = pltpu.make_async_copy(
                zero_ref.at[:, pl.ds(0, pad), :],
                out_ref.at[pl.ds(i, 1), pl.ds(L, pad), :],
                pad_sems.at[i],
            )
            c.start()
            pad_copies.append(c)

    out_copies = []
    for i, L in enumerate(_SEQ_LENS):
        in_copies[i].wait()
        c = pltpu.make_async_copy(
            bufs[i],
            out_ref.at[pl.ds(i, 1), pl.ds(0, L), :],
            out_sems.at[i],
        )
        c.start()
        out_copies.append(c)

    for c in out_copies:
        c.wait()
    for c in pad_copies:
        c.wait()


def kernel(seq0, seq1, seq2, seq3, seq4, seq5, seq6, seq7):
    seqs = (seq0, seq1, seq2, seq3, seq4, seq5, seq6, seq7)
    out_shape = jax.ShapeDtypeStruct((8, _MAX_L, _D), seq0.dtype)
    return pl.pallas_call(
        _body,
        in_specs=[pl.BlockSpec(memory_space=pl.ANY)] * 8,
        out_specs=pl.BlockSpec(memory_space=pl.ANY),
        out_shape=out_shape,
        scratch_shapes=(
            [pltpu.VMEM((1, L, _D), jnp.float32) for L in _SEQ_LENS]
            + [
                pltpu.VMEM((1, _MAX_PAD, _D), jnp.float32),
                pltpu.SemaphoreType.DMA((8,)),
                pltpu.SemaphoreType.DMA((8,)),
                pltpu.SemaphoreType.DMA((8,)),
            ]
        ),
    )(*seqs)
